# baseline (speedup 1.0000x reference)
import functools
import jax
import jax.numpy as jnp
from jax import lax
from jax.experimental import pallas as pl
from jax.experimental.pallas import tpu as pltpu
from jax.experimental.pallas import tpu_sc as plsc

D_MODEL = 64
VOCAB_N = 1000000
SCALE = 8.0
LANES = 16

NUM_CORES = 2
NUM_SUBCORES = 16
NW = NUM_CORES * NUM_SUBCORES

B1, B2 = 4096, 200
N_FULL_TILES = VOCAB_N // 128
TAIL = VOCAB_N - N_FULL_TILES * 128
CB = 1

_MESH = plsc.VectorSubcoreMesh(core_axis_name="c", subcore_axis_name="s")


def _iota16():
    return lax.iota(jnp.int32, LANES)


def _bcast16(v):
    return jax.lax.broadcast(v, (LANES,))


@functools.partial(
    pl.kernel,
    out_type=jax.ShapeDtypeStruct((VOCAB_N, 128), jnp.float32),
    mesh=_MESH,
    scratch_types=[
        pltpu.VMEM((D_MODEL, 128), jnp.float32),
        pltpu.VMEM((128, 128), jnp.float32),
    ],
    compiler_params=pltpu.CompilerParams(use_tc_tiling_on_sc=True, needs_layout_passes=False),
)
def _build_table(lutt_hbm, tailp_hbm, dbl_hbm, tbuf, rbuf):
    wid = lax.axis_index("s") * NUM_CORES + lax.axis_index("c")
    iota = _iota16()

    def transpose_cols(ncols):
        @pl.loop(0, ncols)
        def _(vl):
            vl16 = _bcast16(vl)
            for j in range(D_MODEL // LANES):
                v = plsc.load_gather(tbuf, [j * LANES + iota, vl16]) * SCALE
                rbuf[vl, pl.ds(j * LANES, LANES)] = v
                rbuf[vl, pl.ds(D_MODEL + j * LANES, LANES)] = v

    n_i = 244 + jnp.where(wid < 4, 1, 0).astype(jnp.int32)

    @pl.loop(0, n_i)
    def _(i):
        t = wid + i * NW
        pltpu.sync_copy(lutt_hbm.at[pl.ds(0, D_MODEL), pl.ds(t * 128, 128)], tbuf)
        transpose_cols(128)
        pltpu.sync_copy(rbuf, dbl_hbm.at[pl.ds(t * 128, 128), pl.ds(0, 128)])

    @pl.when(wid == 4)
    def _():
        pltpu.sync_copy(tailp_hbm, tbuf)
        transpose_cols(TAIL)
        pltpu.sync_copy(
            rbuf.at[pl.ds(0, TAIL), pl.ds(0, 128)],
            dbl_hbm.at[pl.ds(N_FULL_TILES * 128, TAIL), pl.ds(0, 128)],
        )


@functools.partial(
    pl.kernel,
    out_type=jax.ShapeDtypeStruct((B2, 8, NW, 8, 128), jnp.float32),
    mesh=_MESH,
    scratch_types=[
        pltpu.VMEM((B2, 128), jnp.int32),
        pltpu.VMEM((128, 128), jnp.float32),
        pltpu.VMEM((8, 8, 128), jnp.float32),
        pltpu.SemaphoreType.DMA,
    ],
    compiler_params=pltpu.CompilerParams(use_tc_tiling_on_sc=True, needs_layout_passes=False),
)
def _gather(xt_hbm, dbl_hbm, out_hbm, idx_all, gbuf, tbuf, sem):
    wid = lax.axis_index("s") * NUM_CORES + lax.axis_index("c")
    iota = _iota16()
    pltpu.sync_copy(xt_hbm.at[pl.ds(0, B2), pl.ds(wid * 128, 128)], idx_all)

    @pl.loop(0, B2 // CB)
    def _(c):
        pltpu.async_copy(dbl_hbm.at[idx_all.at[c]], gbuf, sem).wait()

        @pl.loop(0, 8)
        def _(dh):
            for dl in range(8):
                d16 = _bcast16(dh * 8 + dl)
                for k in range(128 // LANES):
                    v = plsc.load_gather(gbuf, [k * LANES + iota, d16])
                    tbuf[dh, dl, pl.ds(k * LANES, LANES)] = v

        pltpu.sync_copy(
            tbuf, out_hbm.at[c, pl.ds(0, 8), wid]
        )


def kernel(x, lut):
    tail_p = jnp.pad(
        lut[N_FULL_TILES * 128 :].T, ((0, 0), (0, 128 - TAIL))
    )
    dbl = _build_table(lut.T, tail_p)
    out5 = _gather(x.T, dbl)
    return out5.transpose(2, 4, 0, 1, 3).reshape(B1, B2, D_MODEL)


# speedup vs baseline: 2.5020x; 2.5020x over previous
import functools
import jax
import jax.numpy as jnp
from jax import lax
from jax.experimental import pallas as pl
from jax.experimental.pallas import tpu as pltpu
from jax.experimental.pallas import tpu_sc as plsc

D_MODEL = 64
VOCAB_N = 1000000
SCALE = 8.0
LANES = 16

NUM_CORES = 2
NUM_SUBCORES = 16
NW = NUM_CORES * NUM_SUBCORES

B1, B2 = 4096, 200
N_FULL_TILES = VOCAB_N // 128
TAIL = VOCAB_N - N_FULL_TILES * 128

_MESH = plsc.VectorSubcoreMesh(core_axis_name="c", subcore_axis_name="s")


def _iota16():
    return lax.iota(jnp.int32, LANES)


def _bcast16(v):
    return jax.lax.broadcast(v, (LANES,))


@functools.partial(
    pl.kernel,
    out_type=jax.ShapeDtypeStruct((VOCAB_N // 2, 128), jnp.float32),
    mesh=_MESH,
    scratch_types=[
        pltpu.VMEM((D_MODEL, 128), jnp.float32),
        pltpu.VMEM((D_MODEL, 128), jnp.float32),
        pltpu.VMEM((D_MODEL, 128), jnp.float32),
        pltpu.VMEM((D_MODEL, 128), jnp.float32),
        pltpu.SemaphoreType.DMA,
        pltpu.SemaphoreType.DMA,
        pltpu.SemaphoreType.DMA,
        pltpu.SemaphoreType.DMA,
    ],
    compiler_params=pltpu.CompilerParams(
        use_tc_tiling_on_sc=True, needs_layout_passes=False
    ),
)
def _build_table(
    lutt_hbm, tailp_hbm, tab_hbm, tbuf0, tbuf1, rbuf0, rbuf1, rs0, rs1, ws0, ws1
):
    wid = lax.axis_index("s") * NUM_CORES + lax.axis_index("c")
    iota = _iota16()
    tb, rb = (tbuf0, tbuf1), (rbuf0, rbuf1)
    rs, ws = (rs0, rs1), (ws0, ws1)

    def src(t):
        return lutt_hbm.at[pl.ds(0, D_MODEL), pl.ds(t * 128, 128)]

    def dst(t):
        return tab_hbm.at[pl.ds(t * 64, 64), pl.ds(0, 128)]

    def transpose_cols(t_ref, r_ref, npairs):
        @plsc.parallel_loop(0, npairs, unroll=8)
        def _(q):
            for h in range(2):
                vl16 = _bcast16(2 * q + h)
                for j in range(D_MODEL // LANES):
                    v = plsc.load_gather(t_ref, [j * LANES + iota, vl16])
                    r_ref[q, pl.ds(h * D_MODEL + j * LANES, LANES)] = v * SCALE

    n_i = 244 + jnp.where(wid < 4, 1, 0).astype(jnp.int32)
    tile = lambda q: wid + q * NW

    pltpu.async_copy(src(tile(0)), tb[0], rs[0])

    @pl.loop(0, 246, step=2)
    def _(i):
        for b in (0, 1):
            q = i + b

            @pl.when(q < n_i)
            def _():
                @pl.when(q + 1 < n_i)
                def _():
                    pltpu.async_copy(src(tile(q + 1)), tb[1 - b], rs[1 - b])

                pltpu.make_async_copy(src(tile(q)), tb[b], rs[b]).wait()

                @pl.when(q >= 2)
                def _():
                    pltpu.make_async_copy(rb[b], dst(tile(q)), ws[b]).wait()

                transpose_cols(tb[b], rb[b], 64)
                pltpu.async_copy(rb[b], dst(tile(q)), ws[b])

    for b in (0, 1):
        pltpu.make_async_copy(rb[b], dst(tile(0)), ws[b]).wait()

    @pl.when(wid == 4)
    def _():
        pltpu.sync_copy(tailp_hbm, tb[0])
        transpose_cols(tb[0], rb[0], TAIL // 2)
        pltpu.sync_copy(
            rb[0].at[pl.ds(0, TAIL // 2), pl.ds(0, 128)],
            tab_hbm.at[pl.ds(N_FULL_TILES * 64, TAIL // 2), pl.ds(0, 128)],
        )


@functools.partial(
    pl.kernel,
    out_type=jax.ShapeDtypeStruct((B2, 8, NW, 8, 128), jnp.float32),
    mesh=_MESH,
    scratch_types=[
        pltpu.VMEM((B2, 128), jnp.int32),
        pltpu.VMEM((128, D_MODEL), jnp.float32),
        pltpu.VMEM((128, D_MODEL), jnp.float32),
        pltpu.VMEM((8, 8, 128), jnp.float32),
        pltpu.VMEM((8, 8, 128), jnp.float32),
        pltpu.SemaphoreType.DMA,
        pltpu.SemaphoreType.DMA,
        pltpu.SemaphoreType.DMA,
        pltpu.SemaphoreType.DMA,
    ],
    compiler_params=pltpu.CompilerParams(
        use_tc_tiling_on_sc=False, needs_layout_passes=False
    ),
)
def _gather(
    xt_hbm, tab_hbm, out_hbm, idx_all, gbuf0, gbuf1, tbuf0, tbuf1, gs0, gs1, ss0, ss1
):
    wid = lax.axis_index("s") * NUM_CORES + lax.axis_index("c")
    iota = _iota16()
    gb, tb = (gbuf0, gbuf1), (tbuf0, tbuf1)
    gs, ss = (gs0, gs1), (ss0, ss1)
    pltpu.sync_copy(xt_hbm.at[pl.ds(0, B2), wid], idx_all)

    def out_at(c):
        return out_hbm.at[c, pl.ds(0, 8), wid]

    pltpu.async_copy(tab_hbm.at[idx_all.at[0]], gb[0], gs[0])

    @pl.loop(0, B2, step=2)
    def _(cbase):
        for b in (0, 1):
            c = cbase + b

            @pl.when(c + 1 < B2)
            def _():
                pltpu.async_copy(tab_hbm.at[idx_all.at[c + 1]], gb[1 - b], gs[1 - b])

            pltpu.make_async_copy(tab_hbm.at[idx_all.at[c]], gb[b], gs[b]).wait()

            @pl.when(c >= 2)
            def _():
                pltpu.make_async_copy(tb[b], out_at(c), ss[b]).wait()

            @plsc.parallel_loop(0, D_MODEL, unroll=8)
            def _(d):
                d16 = _bcast16(d)
                for k in range(128 // LANES):
                    v = plsc.load_gather(gb[b], [k * LANES + iota, d16])
                    tb[b][d // 8, d % 8, pl.ds(k * LANES, LANES)] = v

            pltpu.async_copy(tb[b], out_at(c), ss[b])

    for b in (0, 1):
        pltpu.make_async_copy(tb[b], out_at(0), ss[b]).wait()


def kernel(x, lut):
    tail_p = jnp.pad(lut[N_FULL_TILES * 128 :].T, ((0, 0), (0, 128 - TAIL)))
    tab = _build_table(lut.T, tail_p)
    tab_lin = tab.reshape(VOCAB_N, D_MODEL)
    xt = x.T.reshape(B2, NW, 128)
    out5 = _gather(xt, tab_lin)
    return out5.transpose(2, 4, 0, 1, 3).reshape(B1, B2, D_MODEL)
